# Initial kernel scaffold; baseline (speedup 1.0000x reference)
import functools

import jax
import jax.numpy as jnp
from jax import lax
from jax.experimental import pallas as pl
from jax.experimental.pallas import tpu as pltpu
from jax.experimental.pallas import tpu_sc as plsc

UT = 130305
NROWS = 96
ROWS_PER_TILE = 3
BLK = 256
RING = 2 * BLK
NFULL = UT // BLK
TAIL = UT - NFULL * BLK


def _make_sc_call():
  mesh = plsc.VectorSubcoreMesh(core_axis_name="c", subcore_axis_name="s")

  @functools.partial(
      pl.kernel,
      out_type=jax.ShapeDtypeStruct((NROWS, UT), jnp.float32),
      mesh=mesh,
      scratch_types=[
          pltpu.VMEM((UT,), jnp.float32),
          pltpu.VMEM((RING,), jnp.float32),
          pltpu.SemaphoreType.DMA,
          pltpu.SemaphoreType.DMA,
      ],
  )
  def sc_permute(x_hbm, out_hbm, row_v, ring_v, sem0, sem1):
    wid = lax.axis_index("s") * 2 + lax.axis_index("c")
    lane = lax.iota(jnp.int32, 16)
    jl = 496 + lane
    c_vec = ((511 - jl) * (510 + jl) >> 1) + jl - 2

    def fire_block(row, bi):
      slot = bi & 1

      @pl.when(bi >= 2)
      def _wait():
        @pl.when(slot == 0)
        def _():
          pltpu.make_async_copy(
              ring_v.at[pl.ds(0, BLK)],
              out_hbm.at[row, pl.ds(0, BLK)], sem0).wait()

        @pl.when(slot == 1)
        def _():
          pltpu.make_async_copy(
              ring_v.at[pl.ds(BLK, BLK)],
              out_hbm.at[row, pl.ds(0, BLK)], sem1).wait()

      @pl.when(slot == 0)
      def _():
        pltpu.async_copy(
            ring_v.at[pl.ds(0, BLK)],
            out_hbm.at[row, pl.ds(bi * BLK, BLK)], sem0)

      @pl.when(slot == 1)
      def _():
        pltpu.async_copy(
            ring_v.at[pl.ds(BLK, BLK)],
            out_hbm.at[row, pl.ds(bi * BLK, BLK)], sem1)

    def do_row(m, _):
      row = wid * ROWS_PER_TILE + m
      pltpu.sync_copy(x_hbm.at[row], row_v)

      def out_row(i, b):
        st_i = (i * (1021 - i)) >> 1
        length = 510 - i
        jv = (i + 2) + lane
        s_vec = ((511 - jv) * (510 + jv) >> 1) + lane
        d_vec = -104 - 16 * jv
        pos0 = (st_i + lane) & (RING - 1)

        nb = (509 - i) >> 4

        def chunk(_, carry):
          s, d, p = carry
          vals = plsc.load_gather(row_v, [s])
          plsc.store_scatter(ring_v, [p], vals)
          return (s + d, d - 256, (p + 16) & (RING - 1))

        lax.fori_loop(0, nb, chunk, (s_vec, d_vec, pos0), unroll=2)

        vals = plsc.load_gather(row_v, [c_vec - i])
        pos_l = ((st_i + length - 16) + lane) & (RING - 1)
        plsc.store_scatter(ring_v, [pos_l], vals, mask=lane >= i - 494)

        b_tgt = (st_i + length) >> 8

        @pl.when(b_tgt >= b + 1)
        def _():
          fire_block(row, b)

        @pl.when(b_tgt >= b + 2)
        def _():
          fire_block(row, b + 1)

        return b_tgt

      lax.fori_loop(0, 510, out_row, 0)

      pltpu.make_async_copy(
          ring_v.at[pl.ds(BLK, BLK)],
          out_hbm.at[row, pl.ds(0, BLK)], sem1).wait()
      tail_pos = (NFULL * BLK) & (RING - 1)
      pltpu.async_copy(
          ring_v.at[pl.ds(tail_pos, TAIL)],
          out_hbm.at[row, pl.ds(NFULL * BLK, TAIL)], sem1)
      pltpu.make_async_copy(
          ring_v.at[pl.ds(0, BLK)],
          out_hbm.at[row, pl.ds(0, BLK)], sem0).wait()
      pltpu.make_async_copy(
          ring_v.at[pl.ds(tail_pos, TAIL)],
          out_hbm.at[row, pl.ds(0, TAIL)], sem1).wait()
      return 0

    lax.fori_loop(0, ROWS_PER_TILE, do_row, 0)

  return sc_permute


_SC_PERMUTE = _make_sc_call()


def kernel(x, reverse):
  bs, num, ut = x.shape

  def do_reverse(xx):
    flat = xx.reshape(NROWS, UT)
    out = _SC_PERMUTE(flat)
    return out.reshape(bs, num, ut)

  return lax.cond(jnp.asarray(reverse) != 0, do_reverse, lambda xx: xx, x)


# speedup vs baseline: 2.2188x; 2.2188x over previous
import functools

import jax
import jax.numpy as jnp
from jax import lax
from jax.experimental import pallas as pl
from jax.experimental.pallas import tpu as pltpu
from jax.experimental.pallas import tpu_sc as plsc

UT = 130305
NROWS = 96
ROWS_PER_TILE = 3
BLK = 128
NSLOT = 4
RING = NSLOT * BLK
NB = (UT - 1) // BLK


def _make_sc_call():
  mesh = plsc.VectorSubcoreMesh(core_axis_name="c", subcore_axis_name="s")

  @functools.partial(
      pl.kernel,
      out_type=jax.ShapeDtypeStruct((NROWS, UT), jnp.float32),
      mesh=mesh,
      compiler_params=pltpu.CompilerParams(needs_layout_passes=False),
      scratch_types=[
          pltpu.VMEM((UT,), jnp.float32),
          pltpu.VMEM((RING,), jnp.float32),
          pltpu.SemaphoreType.DMA,
          pltpu.SemaphoreType.DMA,
          pltpu.SemaphoreType.DMA,
          pltpu.SemaphoreType.DMA,
      ],
  )
  def sc_permute(x_hbm, out_hbm, row_v, ring_v, sem0, sem1, sem2, sem3):
    sems = (sem0, sem1, sem2, sem3)
    wid = lax.axis_index("s") * 2 + lax.axis_index("c")
    lane = lax.iota(jnp.int32, 16)
    jl = 496 + lane
    c_vec = ((511 - jl) * (510 + jl) >> 1) + jl - 2

    def do_row(m, _):
      row = wid * ROWS_PER_TILE + m
      pltpu.sync_copy(x_hbm.at[row], row_v)

      def wait_slot(ws):
        for k in range(NSLOT):
          @pl.when(ws == k)
          def _(k=k):
            pltpu.make_async_copy(
                ring_v.at[pl.ds(k * BLK, BLK)],
                out_hbm.at[row, pl.ds(0, BLK)], sems[k]).wait()

      def fire_block(bi):
        slot = bi & (NSLOT - 1)

        @pl.when(bi >= 2)
        def _():
          wait_slot(slot ^ 2)

        for k in range(NSLOT):
          @pl.when(slot == k)
          def _(k=k):
            pltpu.async_copy(
                ring_v.at[pl.ds(k * BLK, BLK)],
                out_hbm.at[row, pl.ds(bi * BLK, BLK)], sems[k])

      def maybe_fire(b, optr_next):
        tgt = optr_next >> 7

        @pl.when(tgt > b)
        def _():
          fire_block(b)

        return lax.max(b, tgt)

      def out_row(i, b):
        st_i = (i * (1021 - i)) >> 1
        length = 510 - i
        jv = (i + 2) + lane
        s_vec = ((511 - jv) * (510 + jv) >> 1) + lane
        d_vec = -104 - 16 * jv
        pos0 = (st_i + lane) & (RING - 1)

        nb = (509 - i) >> 4

        def chunk(t, carry):
          s, d, p, bb = carry
          vals = plsc.load_gather(row_v, [s])
          plsc.store_scatter(ring_v, [p], vals)
          bb = maybe_fire(bb, st_i + t * 16 + 16)
          return (s + d, d - 256, (p + 16) & (RING - 1), bb)

        _, _, _, b = lax.fori_loop(0, nb, chunk, (s_vec, d_vec, pos0, b))

        vals = plsc.load_gather(row_v, [c_vec - i])
        pos_l = ((st_i + length - 16) + lane) & (RING - 1)
        plsc.store_scatter(ring_v, [pos_l], vals, mask=lane >= i - 494)
        return maybe_fire(b, st_i + length)

      lax.fori_loop(0, 509, out_row, 0)

      pltpu.make_async_copy(
          ring_v.at[pl.ds(0, BLK)],
          out_hbm.at[row, pl.ds(0, BLK)], sem0).wait()
      pltpu.make_async_copy(
          ring_v.at[pl.ds(BLK, BLK)],
          out_hbm.at[row, pl.ds(0, BLK)], sem1).wait()
      return 0

    lax.fori_loop(0, ROWS_PER_TILE, do_row, 0)

  return sc_permute


_SC_PERMUTE = _make_sc_call()


def kernel(x, reverse):
  bs, num, ut = x.shape

  def do_reverse(xx):
    flat = xx.reshape(NROWS, UT)
    out = _SC_PERMUTE(flat)
    out = out.at[:, UT - 1].set(flat[:, 0])
    return out.reshape(bs, num, ut)

  return lax.cond(jnp.asarray(reverse) != 0, do_reverse, lambda xx: xx, x)
